# Initial kernel scaffold; baseline (speedup 1.0000x reference)
import jax
import jax.numpy as jnp
from jax.experimental import pallas as pl


def kernel(x, edge_index, W1l, b1l, W1r, g1, be1, W2l, b2l, W2r, g2, be2, W3l, b3l, W3r):
    raise NotImplementedError("write your pallas kernel here")


# speedup vs baseline: 7.7031x; 7.7031x over previous
import functools

import jax
import jax.numpy as jnp
from jax import lax
from jax.experimental import pallas as pl
from jax.experimental.pallas import tpu as pltpu
from jax.experimental.pallas import tpu_sc as plsc

N = 10000
E = 320000
D_IN = 128
D_H = 128
D_OUT = 64
EPS = 1e-5

NC = 2
NS = 16
NW = NC * NS
CHUNK = 320
NCHUNKS = E // CHUNK
BASE_STEPS = NCHUNKS // NW
EXTRA = NCHUNKS % NW
ROWS_PER_S = 624
TAIL0 = ROWS_PER_S * NS
TAIL = N - TAIL0

_MESH = dict(core_axis_name="c", subcore_axis_name="s",
             num_cores=NC, num_subcores=NS)


@functools.lru_cache(maxsize=None)
def _make_seg_sum(D):
  @functools.partial(
      pl.kernel,
      mesh=plsc.VectorSubcoreMesh(**_MESH),
      out_type=jax.ShapeDtypeStruct((NC, N, D), jnp.float32),
      scratch_types=[
          pltpu.VMEM((CHUNK,), jnp.int32),
          pltpu.VMEM((CHUNK,), jnp.int32),
          pltpu.VMEM((CHUNK, D), jnp.float32),
          pltpu.VMEM_SHARED((N, D), jnp.float32),
          pltpu.SemaphoreType.DMA,
      ],
  )
  def seg_sum(table_hbm, src_hbm, dst_hbm, zeros_hbm, out_hbm,
              idx_s, idx_d, rows, acc, sem):
    cid = lax.axis_index("c")
    sid = lax.axis_index("s")
    wid = cid * NS + sid
    row0 = sid * ROWS_PER_S

    pltpu.sync_copy(zeros_hbm, acc.at[pl.ds(row0, ROWS_PER_S)])
    @pl.when(sid == 0)
    def _():
      pltpu.sync_copy(zeros_hbm.at[pl.ds(0, TAIL)], acc.at[pl.ds(TAIL0, TAIL)])
    plsc.subcore_barrier()

    def step(i, carry):
      base = (wid + i * NW) * CHUNK
      pltpu.sync_copy(src_hbm.at[pl.ds(base, CHUNK)], idx_s)
      pltpu.sync_copy(dst_hbm.at[pl.ds(base, CHUNK)], idx_d)
      pltpu.async_copy(table_hbm.at[idx_s], rows, sem).wait()
      pltpu.sync_copy(rows, acc.at[idx_d], add=True)
      return carry

    n_steps = BASE_STEPS + jnp.where(wid < EXTRA, 1, 0)
    lax.fori_loop(0, n_steps, step, 0)
    plsc.subcore_barrier()

    pltpu.sync_copy(acc.at[pl.ds(row0, ROWS_PER_S)],
                    out_hbm.at[cid, pl.ds(row0, ROWS_PER_S)])
    @pl.when(sid == 0)
    def _():
      pltpu.sync_copy(acc.at[pl.ds(TAIL0, TAIL)],
                      out_hbm.at[cid, pl.ds(TAIL0, TAIL)])

  return seg_sum


@functools.lru_cache(maxsize=None)
def _make_deg():
  @functools.partial(
      pl.kernel,
      mesh=plsc.VectorSubcoreMesh(**_MESH),
      out_type=jax.ShapeDtypeStruct((NC, N, 128), jnp.float32),
      scratch_types=[
          pltpu.VMEM((CHUNK,), jnp.int32),
          pltpu.VMEM((CHUNK, 128), jnp.float32),
          pltpu.VMEM_SHARED((N, 128), jnp.float32),
      ],
  )
  def deg_kernel(dst_hbm, ones_hbm, zeros_hbm, out_hbm, idx_d, ones, acc):
    cid = lax.axis_index("c")
    sid = lax.axis_index("s")
    wid = cid * NS + sid
    row0 = sid * ROWS_PER_S

    pltpu.sync_copy(zeros_hbm, acc.at[pl.ds(row0, ROWS_PER_S)])
    pltpu.sync_copy(ones_hbm, ones)
    @pl.when(sid == 0)
    def _():
      pltpu.sync_copy(zeros_hbm.at[pl.ds(0, TAIL)], acc.at[pl.ds(TAIL0, TAIL)])
    plsc.subcore_barrier()

    def step(i, carry):
      base = (wid + i * NW) * CHUNK
      pltpu.sync_copy(dst_hbm.at[pl.ds(base, CHUNK)], idx_d)
      pltpu.sync_copy(ones, acc.at[idx_d], add=True)
      return carry

    n_steps = BASE_STEPS + jnp.where(wid < EXTRA, 1, 0)
    lax.fori_loop(0, n_steps, step, 0)
    plsc.subcore_barrier()

    pltpu.sync_copy(acc.at[pl.ds(row0, ROWS_PER_S)],
                    out_hbm.at[cid, pl.ds(row0, ROWS_PER_S)])
    @pl.when(sid == 0)
    def _():
      pltpu.sync_copy(acc.at[pl.ds(TAIL0, TAIL)],
                      out_hbm.at[cid, pl.ds(TAIL0, TAIL)])

  return deg_kernel


def _bn_relu(h, g, b):
  m = jnp.mean(h, axis=0)
  d = h - m[None, :]
  v = jnp.mean(d * d, axis=0)
  return jnp.maximum(d * lax.rsqrt(v + EPS)[None, :] * g[None, :] + b[None, :],
                     0.0)


def _matT(a, w):
  return lax.dot_general(a, w, (((1,), (1,)), ((), ())),
                         preferred_element_type=jnp.float32)


def _dense1_body(s_ref, dg_ref, x_ref, wl_ref, bl_ref, wr_ref, g_ref, be_ref,
                 h_out, inv_out):
  deg = dg_ref[0, :, 0:16] + dg_ref[1, :, 0:16]
  inv = 1.0 / jnp.maximum(deg, 1.0)
  inv_out[...] = inv
  agg = (s_ref[0] + s_ref[1]) * inv[:, 0:1]
  h = _matT(agg, wl_ref[...]) + bl_ref[...][None, :] + _matT(x_ref[...], wr_ref[...])
  h_out[...] = _bn_relu(h, g_ref[...], be_ref[...])


def _dense2_body(s_ref, h1_ref, inv_ref, wl_ref, bl_ref, wr_ref, g_ref, be_ref,
                 w3r_ref, h2_out, r_out):
  agg = (s_ref[0] + s_ref[1]) * inv_ref[...][:, 0:1]
  h = _matT(agg, wl_ref[...]) + bl_ref[...][None, :] + _matT(h1_ref[...], wr_ref[...])
  h2 = _bn_relu(h, g_ref[...], be_ref[...])
  h2_out[...] = h2
  r_out[...] = _matT(h2, w3r_ref[...])


def _dense3_body(s_ref, r_ref, inv_ref, w3l_ref, bl_ref, p_out):
  agg = (s_ref[0] + s_ref[1]) * inv_ref[...][:, 0:1]
  logits = _matT(agg, w3l_ref[...]) + bl_ref[...][None, :] + r_ref[...]
  mx = jnp.max(logits, axis=-1, keepdims=True)
  e = jnp.exp(logits - mx)
  p_out[...] = e / jnp.sum(e, axis=-1, keepdims=True)


_dense1 = pl.pallas_call(
    _dense1_body,
    out_shape=[jax.ShapeDtypeStruct((N, D_H), jnp.float32),
               jax.ShapeDtypeStruct((N, 16), jnp.float32)],
)

_dense2 = pl.pallas_call(
    _dense2_body,
    out_shape=[jax.ShapeDtypeStruct((N, D_H), jnp.float32),
               jax.ShapeDtypeStruct((N, D_OUT), jnp.float32)],
)

_dense3 = pl.pallas_call(
    _dense3_body,
    out_shape=jax.ShapeDtypeStruct((N, D_OUT), jnp.float32),
)


def kernel(x, edge_index, W1l, b1l, W1r, g1, be1, W2l, b2l, W2r, g2, be2,
           W3l, b3l, W3r):
  src = edge_index[0]
  dst = edge_index[1]

  z128 = jnp.zeros((ROWS_PER_S, D_H), jnp.float32)
  ones = jnp.ones((CHUNK, 128), jnp.float32)
  seg = _make_seg_sum(D_H)

  dg = _make_deg()(dst, ones, z128)
  s1 = seg(x, src, dst, z128)
  h1, inv = _dense1(s1, dg, x, W1l, b1l, W1r, g1, be1)

  s2 = seg(h1, src, dst, z128)
  h2, r = _dense2(s2, h1, inv, W2l, b2l, W2r, g2, be2, W3r)

  s3 = seg(h2, src, dst, z128)
  return _dense3(s3, r, inv, W3l, b3l)
